# Initial kernel scaffold; baseline (speedup 1.0000x reference)
import jax
import jax.numpy as jnp
from jax.experimental import pallas as pl


def kernel(V_predict, L, V_w, V_compliance, C_dist, C_init_d):
    raise NotImplementedError("write your pallas kernel here")


# speedup vs baseline: 81.6159x; 81.6159x over previous
import functools

import jax
import jax.numpy as jnp
from jax import lax
from jax.experimental import pallas as pl
from jax.experimental.pallas import tpu as pltpu
from jax.experimental.pallas import tpu_sc as plsc

N = 100000
E = 6400000
CE = 2048
SUB = 128
G = CE // SUB
LANES = 16
NGROUP = CE // LANES
NW = 32
NCHUNK = E // CE
BASE_CHUNKS = NCHUNK // NW
EXTRA = NCHUNK - BASE_CHUNKS * NW
RPT = N // 16


def _rsqrt(x):
    i = lax.bitcast_convert_type(x, jnp.int32)
    i = jnp.int32(0x5F3759DF) - lax.shift_right_logical(i, 1)
    y = lax.bitcast_convert_type(i, jnp.float32)
    for _ in range(3):
        y = y * (jnp.float32(1.5) - jnp.float32(0.5) * x * y * y)
    return y


_MESH = plsc.VectorSubcoreMesh(core_axis_name="c", subcore_axis_name="s")


@functools.partial(
    pl.kernel,
    out_type=[
        jax.ShapeDtypeStruct((2, N, 4), jnp.float32),
        jax.ShapeDtypeStruct((E,), jnp.float32),
    ],
    mesh=_MESH,
    compiler_params=pltpu.CompilerParams(
        needs_layout_passes=False, use_tc_tiling_on_sc=False),
    scratch_types=[
        pltpu.VMEM((G, SUB), jnp.int32),
        pltpu.VMEM((G, SUB), jnp.int32),
        pltpu.VMEM((CE, 8), jnp.float32),
        pltpu.VMEM((CE, 8), jnp.float32),
        pltpu.VMEM((CE,), jnp.float32),
        pltpu.VMEM((CE,), jnp.float32),
        pltpu.VMEM((CE,), jnp.float32),
        pltpu.VMEM((CE, 4), jnp.float32),
        pltpu.VMEM((CE, 4), jnp.float32),
        pltpu.VMEM_SHARED((N, 4), jnp.float32),
    ],
)
def _sc_solve(tab, ii, jj, l_in, d0_in, zeros_hbm,
              parts, l_out,
              idx_i, idx_j, rows_i, rows_j, l_v, d0_v, lout_v,
              ci_v, cj_v, acc):
    c = lax.axis_index("c")
    s = lax.axis_index("s")
    w = s * 2 + c

    @pl.when(s == 0)
    def _():
        pltpu.sync_copy(zeros_hbm, acc)

    plsc.subcore_barrier()

    ib = lax.iota(jnp.int32, LANES)
    col = [jnp.full((LANES,), k, jnp.int32) for k in range(5)]
    ccol = [jnp.full((LANES,), k, jnp.int32) for k in range(3)]

    def group_body(g, _):
        rowv = ib + g * LANES
        xi = plsc.load_gather(rows_i, [rowv, col[0]])
        yi = plsc.load_gather(rows_i, [rowv, col[1]])
        zi = plsc.load_gather(rows_i, [rowv, col[2]])
        wi = plsc.load_gather(rows_i, [rowv, col[3]])
        ki = plsc.load_gather(rows_i, [rowv, col[4]])
        xj = plsc.load_gather(rows_j, [rowv, col[0]])
        yj = plsc.load_gather(rows_j, [rowv, col[1]])
        zj = plsc.load_gather(rows_j, [rowv, col[2]])
        wj = plsc.load_gather(rows_j, [rowv, col[3]])
        kj = plsc.load_gather(rows_j, [rowv, col[4]])
        dx = xi - xj
        dy = yi - yj
        dz = zi - zj
        dsq = dx * dx + dy * dy + dz * dz
        rinv = _rsqrt(dsq)
        dist = dsq * rinv
        lv = l_v[pl.ds(g * LANES, LANES)]
        d0v = d0_v[pl.ds(g * LANES, LANES)]
        cons = dist - d0v
        a = (ki + kj) * jnp.float32(0.5)
        ssum = wi + wj
        ssum = jnp.where(ssum == jnp.float32(0.0), jnp.float32(jnp.inf), ssum)
        ldel = (-cons - a * lv) / (ssum + a)
        lout_v[pl.ds(g * LANES, LANES)] = lv + ldel
        rn = jnp.where(dsq > jnp.float32(0.0), rinv, jnp.float32(jnp.inf))
        fi = wi * ldel * rn
        fj = -(wj * ldel * rn)
        plsc.store_scatter(ci_v, [rowv, ccol[0]], fi * dx)
        plsc.store_scatter(ci_v, [rowv, ccol[1]], fi * dy)
        plsc.store_scatter(ci_v, [rowv, ccol[2]], fi * dz)
        plsc.store_scatter(cj_v, [rowv, ccol[0]], fj * dx)
        plsc.store_scatter(cj_v, [rowv, ccol[1]], fj * dy)
        plsc.store_scatter(cj_v, [rowv, ccol[2]], fj * dz)
        return 0

    def chunk_body(t, _):
        chunk = t * NW + w
        ebase = chunk * CE
        pltpu.sync_copy(ii.at[pl.ds(chunk * G, G)], idx_i)
        pltpu.sync_copy(jj.at[pl.ds(chunk * G, G)], idx_j)
        pltpu.sync_copy(l_in.at[pl.ds(ebase, CE)], l_v)
        pltpu.sync_copy(d0_in.at[pl.ds(ebase, CE)], d0_v)

        def gat(g, _):
            pltpu.sync_copy(tab.at[idx_i.at[g]], rows_i.at[pl.ds(g * SUB, SUB)])
            pltpu.sync_copy(tab.at[idx_j.at[g]], rows_j.at[pl.ds(g * SUB, SUB)])
            return 0

        lax.fori_loop(0, G, gat, 0)
        lax.fori_loop(0, NGROUP, group_body, 0)
        pltpu.sync_copy(lout_v, l_out.at[pl.ds(ebase, CE)])

        def scat(g, _):
            pltpu.sync_copy(ci_v.at[pl.ds(g * SUB, SUB)], acc.at[idx_i.at[g]],
                            add=True)
            pltpu.sync_copy(cj_v.at[pl.ds(g * SUB, SUB)], acc.at[idx_j.at[g]],
                            add=True)
            return 0

        lax.fori_loop(0, G, scat, 0)
        return 0

    nchunks = jnp.where(w < EXTRA, BASE_CHUNKS + 1, BASE_CHUNKS)
    lax.fori_loop(0, nchunks, chunk_body, 0)

    plsc.subcore_barrier()

    @pl.when(s == 0)
    def _():
        pltpu.sync_copy(acc, parts.at[c])


def _combine_body(vp_ref, p0_ref, p1_ref, o_ref):
    o_ref[...] = vp_ref[...] + p0_ref[...] + p1_ref[...]


def _combine(v_pad, parts):
    vp = v_pad.reshape(N * 4 // 128, 128)
    p0 = parts[0].reshape(N * 4 // 128, 128)
    p1 = parts[1].reshape(N * 4 // 128, 128)
    out = pl.pallas_call(
        _combine_body,
        out_shape=jax.ShapeDtypeStruct((N * 4 // 128, 128), jnp.float32),
    )(vp, p0, p1)
    return out.reshape(N, 4)[:, :3]


def kernel(V_predict, L, V_w, V_compliance, C_dist, C_init_d):
    tab = jnp.concatenate(
        [V_predict, V_w, V_compliance, jnp.zeros((N, 3), jnp.float32)], axis=1)
    ii = C_dist[:, 0].reshape(NCHUNK * G, SUB)
    jj = C_dist[:, 1].reshape(NCHUNK * G, SUB)
    zeros_hbm = jnp.zeros((N, 4), jnp.float32)
    parts, l_new = _sc_solve(tab, ii, jj, L.reshape(E), C_init_d.reshape(E),
                             zeros_hbm)
    v_pad = jnp.concatenate([V_predict, jnp.zeros((N, 1), jnp.float32)], axis=1)
    v_out = _combine(v_pad, parts)
    return (v_out, l_new.reshape(E, 1))


# speedup vs baseline: 159.2416x; 1.9511x over previous
import functools

import jax
import jax.numpy as jnp
from jax import lax
from jax.experimental import pallas as pl
from jax.experimental.pallas import tpu as pltpu
from jax.experimental.pallas import tpu_sc as plsc

N = 100000
E = 6400000
CE = 2048
SUB = 128
G = CE // SUB
LANES = 16
NGROUP = CE // LANES
NW = 32
NCHUNK = E // CE
BASE_CHUNKS = NCHUNK // NW
EXTRA = NCHUNK - BASE_CHUNKS * NW
RPT = N // 16


def _rsqrt(x):
    i = lax.bitcast_convert_type(x, jnp.int32)
    i = jnp.int32(0x5F3759DF) - lax.shift_right_logical(i, 1)
    y = lax.bitcast_convert_type(i, jnp.float32)
    for _ in range(3):
        y = y * (jnp.float32(1.5) - jnp.float32(0.5) * x * y * y)
    return y


_MESH = plsc.VectorSubcoreMesh(core_axis_name="c", subcore_axis_name="s")


@functools.partial(
    pl.kernel,
    out_type=[
        jax.ShapeDtypeStruct((2, N, 4), jnp.float32),
        jax.ShapeDtypeStruct((E,), jnp.float32),
    ],
    mesh=_MESH,
    compiler_params=pltpu.CompilerParams(
        needs_layout_passes=False, use_tc_tiling_on_sc=False),
    scratch_types=[
        pltpu.VMEM((CE,), jnp.int32),
        pltpu.VMEM((CE,), jnp.int32),
        pltpu.VMEM((CE, 8), jnp.float32),
        pltpu.VMEM((CE, 8), jnp.float32),
        pltpu.VMEM((CE,), jnp.float32),
        pltpu.VMEM((CE,), jnp.float32),
        pltpu.VMEM((CE,), jnp.float32),
        pltpu.VMEM((CE, 4), jnp.float32),
        pltpu.VMEM((CE, 4), jnp.float32),
        pltpu.VMEM_SHARED((N, 4), jnp.float32),
    ],
)
def _sc_solve(tab, ii, jj, l_in, d0_in, zeros_hbm,
              parts, l_out,
              idx_i, idx_j, rows_i, rows_j, l_v, d0_v, lout_v,
              ci_v, cj_v, acc):
    c = lax.axis_index("c")
    s = lax.axis_index("s")
    w = s * 2 + c

    @pl.when(s == 0)
    def _():
        pltpu.sync_copy(zeros_hbm, acc)

    plsc.subcore_barrier()

    ib = lax.iota(jnp.int32, LANES)
    col = [jnp.full((LANES,), k, jnp.int32) for k in range(5)]
    ccol = [jnp.full((LANES,), k, jnp.int32) for k in range(3)]

    def group_body(g, _):
        rowv = ib + g * LANES
        xi = plsc.load_gather(rows_i, [rowv, col[0]])
        yi = plsc.load_gather(rows_i, [rowv, col[1]])
        zi = plsc.load_gather(rows_i, [rowv, col[2]])
        wi = plsc.load_gather(rows_i, [rowv, col[3]])
        ki = plsc.load_gather(rows_i, [rowv, col[4]])
        xj = plsc.load_gather(rows_j, [rowv, col[0]])
        yj = plsc.load_gather(rows_j, [rowv, col[1]])
        zj = plsc.load_gather(rows_j, [rowv, col[2]])
        wj = plsc.load_gather(rows_j, [rowv, col[3]])
        kj = plsc.load_gather(rows_j, [rowv, col[4]])
        dx = xi - xj
        dy = yi - yj
        dz = zi - zj
        dsq = dx * dx + dy * dy + dz * dz
        rinv = _rsqrt(dsq)
        dist = dsq * rinv
        lv = l_v[pl.ds(g * LANES, LANES)]
        d0v = d0_v[pl.ds(g * LANES, LANES)]
        cons = dist - d0v
        a = (ki + kj) * jnp.float32(0.5)
        ssum = wi + wj
        ssum = jnp.where(ssum == jnp.float32(0.0), jnp.float32(jnp.inf), ssum)
        ldel = (-cons - a * lv) / (ssum + a)
        lout_v[pl.ds(g * LANES, LANES)] = lv + ldel
        rn = jnp.where(dsq > jnp.float32(0.0), rinv, jnp.float32(jnp.inf))
        fi = wi * ldel * rn
        fj = -(wj * ldel * rn)
        plsc.store_scatter(ci_v, [rowv, ccol[0]], fi * dx)
        plsc.store_scatter(ci_v, [rowv, ccol[1]], fi * dy)
        plsc.store_scatter(ci_v, [rowv, ccol[2]], fi * dz)
        plsc.store_scatter(cj_v, [rowv, ccol[0]], fj * dx)
        plsc.store_scatter(cj_v, [rowv, ccol[1]], fj * dy)
        plsc.store_scatter(cj_v, [rowv, ccol[2]], fj * dz)
        return 0

    def chunk_body(t, _):
        chunk = t * NW + w
        ebase = chunk * CE
        pltpu.sync_copy(ii.at[pl.ds(ebase, CE)], idx_i)
        pltpu.sync_copy(jj.at[pl.ds(ebase, CE)], idx_j)
        pltpu.sync_copy(l_in.at[pl.ds(ebase, CE)], l_v)
        pltpu.sync_copy(d0_in.at[pl.ds(ebase, CE)], d0_v)
        pltpu.sync_copy(tab.at[idx_i], rows_i)
        pltpu.sync_copy(tab.at[idx_j], rows_j)
        lax.fori_loop(0, NGROUP, group_body, 0)
        pltpu.sync_copy(lout_v, l_out.at[pl.ds(ebase, CE)])
        pltpu.sync_copy(ci_v, acc.at[idx_i], add=True)
        pltpu.sync_copy(cj_v, acc.at[idx_j], add=True)
        return 0

    nchunks = jnp.where(w < EXTRA, BASE_CHUNKS + 1, BASE_CHUNKS)
    lax.fori_loop(0, nchunks, chunk_body, 0)

    plsc.subcore_barrier()

    @pl.when(s == 0)
    def _():
        pltpu.sync_copy(acc, parts.at[c])


def _combine_body(vp_ref, p0_ref, p1_ref, o_ref):
    o_ref[...] = vp_ref[...] + p0_ref[...] + p1_ref[...]


def _combine(v_pad, parts):
    vp = v_pad.reshape(N * 4 // 128, 128)
    p0 = parts[0].reshape(N * 4 // 128, 128)
    p1 = parts[1].reshape(N * 4 // 128, 128)
    out = pl.pallas_call(
        _combine_body,
        out_shape=jax.ShapeDtypeStruct((N * 4 // 128, 128), jnp.float32),
    )(vp, p0, p1)
    return out.reshape(N, 4)[:, :3]


def kernel(V_predict, L, V_w, V_compliance, C_dist, C_init_d):
    tab = jnp.concatenate(
        [V_predict, V_w, V_compliance, jnp.zeros((N, 3), jnp.float32)], axis=1)
    ii = C_dist[:, 0]
    jj = C_dist[:, 1]
    zeros_hbm = jnp.zeros((N, 4), jnp.float32)
    parts, l_new = _sc_solve(tab, ii, jj, L.reshape(E), C_init_d.reshape(E),
                             zeros_hbm)
    v_pad = jnp.concatenate([V_predict, jnp.zeros((N, 1), jnp.float32)], axis=1)
    v_out = _combine(v_pad, parts)
    return (v_out, l_new.reshape(E, 1))
